# Initial kernel scaffold; baseline (speedup 1.0000x reference)
import jax
import jax.numpy as jnp
from jax.experimental import pallas as pl


def kernel(z, batch, W1, b1, W2, b2, W3, b3):
    raise NotImplementedError("write your pallas kernel here")


# speedup vs baseline: 4.0213x; 4.0213x over previous
import functools

import jax
import jax.numpy as jnp
from jax import lax
from jax.experimental import pallas as pl
from jax.experimental.pallas import tpu as pltpu
import jax.experimental.pallas.tpu_sc as plsc

N = 100000
NSEG = 1024
LAT = 128
NC, NS = 2, 16
NW = NC * NS
BLK = 128
NFULL = N // BLK
TAIL = N - NFULL * BLK
NBLK = NFULL + 1
TRIPS = (NBLK + NW - 1) // NW
ZR = NSEG // NS
CW = 16

_f32 = jnp.float32


def _sc_body(z_hbm, ids_hbm, part_out, cnt_out,
             rows_v, trows_v, idx_v, tidx_v, ones_v, zrow_v, zc_v,
             acc_s, cnt_s):
    cid = lax.axis_index("c")
    sid = lax.axis_index("s")
    wid = sid * NC + cid

    def _fill_rows(i, _):
        zrow_v[i // 8, pl.ds((i % 8) * 16, 16)] = jnp.zeros((16,), _f32)
        return 0
    lax.fori_loop(0, ZR * 8, _fill_rows, 0)

    def _fill_zc(i, _):
        zc_v[i, :] = jnp.zeros((16,), _f32)
        return 0
    lax.fori_loop(0, ZR, _fill_zc, 0)

    def _fill_ones(i, _):
        ones_v[i, :] = jnp.ones((16,), _f32)
        return 0
    lax.fori_loop(0, BLK, _fill_ones, 0)

    pltpu.sync_copy(zrow_v, acc_s.at[pl.ds(sid * ZR, ZR), :])
    pltpu.sync_copy(zc_v, cnt_s.at[pl.ds(sid * ZR, ZR), :])
    plsc.subcore_barrier()

    def _trip(t, _):
        blk = wid + NW * t

        @pl.when(blk < NFULL)
        def _full():
            pltpu.sync_copy(ids_hbm.at[blk], idx_v)
            pltpu.sync_copy(z_hbm.at[pl.ds(blk * BLK, BLK), :], rows_v)
            pltpu.sync_copy(rows_v, acc_s.at[idx_v], add=True)
            pltpu.sync_copy(ones_v, cnt_s.at[idx_v], add=True)

        @pl.when(blk == NFULL)
        def _tail():
            pltpu.sync_copy(ids_hbm.at[NFULL, pl.ds(0, TAIL)], tidx_v)
            pltpu.sync_copy(z_hbm.at[pl.ds(NFULL * BLK, TAIL), :], trows_v)
            pltpu.sync_copy(trows_v, acc_s.at[tidx_v], add=True)
            pltpu.sync_copy(ones_v.at[pl.ds(0, TAIL), :], cnt_s.at[tidx_v],
                            add=True)

        return 0

    lax.fori_loop(0, TRIPS, _trip, 0)
    plsc.subcore_barrier()

    pltpu.sync_copy(acc_s.at[pl.ds(sid * ZR, ZR), :],
                    part_out.at[cid, pl.ds(sid * ZR, ZR), :])
    pltpu.sync_copy(cnt_s.at[pl.ds(sid * ZR, ZR), :],
                    cnt_out.at[cid, pl.ds(sid * ZR, ZR), :])


_sc_segment_sum = functools.partial(
    pl.kernel,
    out_type=(jax.ShapeDtypeStruct((NC, NSEG, LAT), _f32),
              jax.ShapeDtypeStruct((NC, NSEG, CW), _f32)),
    mesh=plsc.VectorSubcoreMesh(core_axis_name="c", subcore_axis_name="s",
                                num_cores=NC, num_subcores=NS),
    scratch_types=[
        pltpu.VMEM((BLK, LAT), _f32),
        pltpu.VMEM((TAIL, LAT), _f32),
        pltpu.VMEM((BLK,), jnp.int32),
        pltpu.VMEM((TAIL,), jnp.int32),
        pltpu.VMEM((BLK, CW), _f32),
        pltpu.VMEM((ZR, LAT), _f32),
        pltpu.VMEM((ZR, CW), _f32),
        pltpu.VMEM_SHARED((NSEG, LAT), _f32),
        pltpu.VMEM_SHARED((NSEG, CW), _f32),
    ],
)(_sc_body)


def _mlp_body(part_ref, cnt_ref, w1_ref, b1_ref, w2_ref, b2_ref, w3_ref,
              b3_ref, out_ref):
    summed = part_ref[0] + part_ref[1]
    c = cnt_ref[0] + cnt_ref[1]
    counts = c[:, 0:1]
    pooled = summed / counts
    h = jnp.dot(pooled, w1_ref[...], preferred_element_type=_f32) + b1_ref[...]
    h = h * (1.0 / (1.0 + jnp.exp(-h)))
    h = jnp.dot(h, w2_ref[...], preferred_element_type=_f32) + b2_ref[...]
    h = h * (1.0 / (1.0 + jnp.exp(-h)))
    out = jnp.dot(h, w3_ref[...], preferred_element_type=_f32) + b3_ref[...]
    out_ref[...] = out


_mlp = pl.pallas_call(
    _mlp_body,
    out_shape=jax.ShapeDtypeStruct((NSEG, 1), _f32),
)


def kernel(z, batch, W1, b1, W2, b2, W3, b3):
    ids = batch.astype(jnp.int32)
    pad = NBLK * BLK - N
    ids2d = jnp.concatenate([ids, jnp.zeros((pad,), jnp.int32)]).reshape(
        NBLK, BLK)
    part, cnt = _sc_segment_sum(z, ids2d)
    out = _mlp(part, cnt, W1, b1.reshape(1, LAT), W2, b2.reshape(1, 64),
               W3, b3.reshape(1, 1))
    return out[:, 0]


# speedup vs baseline: 5.3539x; 1.3314x over previous
import functools

import jax
import jax.numpy as jnp
from jax import lax
from jax.experimental import pallas as pl
from jax.experimental.pallas import tpu as pltpu
import jax.experimental.pallas.tpu_sc as plsc

N = 100000
NSEG = 1024
LAT = 128
NC, NS = 2, 16
NW = NC * NS
BLK = 128
NFULL = N // BLK
TAIL = N - NFULL * BLK
TRIPS = 25
BASE_Q, BASE_R = divmod(NFULL, NW)
ZR = NSEG // NS
CW = 16
ROWW = LAT + CW
NBUF = 4
LOOKAHEAD = 3

_f32 = jnp.float32


def _sc_body(z_hbm, ids_hbm, part_out, cnt_out,
             bufs, trows_v, idx_all, tidx_v, zrow_v, cntv_v, acc_s,
             isem, lsems, ssems):
    cid = lax.axis_index("c")
    sid = lax.axis_index("s")
    wid = sid * NC + cid
    nb = jnp.where(wid < BASE_R, BASE_Q + 1, BASE_Q)
    base = wid * BASE_Q + jnp.minimum(wid, BASE_R)

    ones16 = jnp.ones((16,), _f32)
    for b in range(NBUF):
        def _fill_ones(r, _, b=b):
            bufs[b][r, pl.ds(LAT, CW)] = ones16
            return 0
        lax.fori_loop(0, BLK, _fill_ones, 0)

    def _fill_tail_ones(r, _):
        trows_v[r, pl.ds(LAT, CW)] = ones16
        return 0
    lax.fori_loop(0, TAIL, _fill_tail_ones, 0)

    NCH = ROWW // 16

    def _fill_zero(i, _):
        zrow_v[i // NCH, pl.ds((i % NCH) * 16, 16)] = jnp.zeros((16,), _f32)
        return 0
    lax.fori_loop(0, ZR * NCH, _fill_zero, 0)

    base_off = pl.multiple_of(base * BLK, BLK)

    def _load_desc(t, j):
        off = pl.multiple_of(base_off + t * BLK, 8)
        return pltpu.make_async_copy(z_hbm.at[pl.ds(off, BLK), :],
                                     bufs[j].at[:, pl.ds(0, LAT)], lsems[j])

    def _scat_desc(t, j):
        return pltpu.make_async_copy(bufs[j], acc_s.at[idx_all.at[t]],
                                     ssems[j])

    ids_desc = [
        pltpu.make_async_copy(
            ids_hbm.at[pl.ds(pl.multiple_of(base_off + t * BLK, 8), BLK)],
            idx_all.at[t], isem)
        for t in range(TRIPS)
    ]
    for t in range(TRIPS):
        ids_desc[t].start()
    for t in range(LOOKAHEAD):
        _load_desc(t, t % NBUF).start()

    pltpu.sync_copy(zrow_v, acc_s.at[pl.ds(sid * ZR, ZR), :])
    plsc.subcore_barrier()
    for t in range(TRIPS):
        ids_desc[t].wait()

    NGRP = (TRIPS + NBUF) // NBUF + (1 if (TRIPS + NBUF) % NBUF else 0)

    def _group(g, _):
        t0 = g * NBUF
        for j in range(NBUF):
            t = t0 + j

            @pl.when(jnp.logical_and(t >= 1, t - 1 < nb))
            def _():
                _scat_desc(t - 1, (j - 1) % NBUF).wait()

            u = t + LOOKAHEAD

            @pl.when(u < nb)
            def _():
                _load_desc(u, (j + LOOKAHEAD) % NBUF).start()

            @pl.when(t < nb)
            def _():
                _load_desc(t, j).wait()
                _scat_desc(t, j).start(add=True)

        return 0

    lax.fori_loop(0, NGRP, _group, 0)

    @pl.when(wid == NW - 1)
    def _tail():
        pltpu.sync_copy(ids_hbm.at[pl.ds(NFULL * BLK, TAIL)], tidx_v)
        pltpu.sync_copy(z_hbm.at[pl.ds(NFULL * BLK, TAIL), :],
                        trows_v.at[:, pl.ds(0, LAT)])
        pltpu.sync_copy(trows_v, acc_s.at[tidx_v], add=True)

    plsc.subcore_barrier()

    pltpu.sync_copy(acc_s.at[pl.ds(sid * ZR, ZR), pl.ds(0, LAT)],
                    part_out.at[cid, pl.ds(sid * ZR, ZR), :])
    pltpu.sync_copy(acc_s.at[pl.ds(sid * ZR, ZR), pl.ds(LAT, CW)], cntv_v)

    def _bcast(r, _):
        row = jnp.full((16,), cntv_v[r, :][0], _f32)
        for k in range(LAT // 16):
            zrow_v[r, pl.ds(k * 16, 16)] = row
        return 0

    lax.fori_loop(0, ZR, _bcast, 0)
    pltpu.sync_copy(zrow_v.at[:, pl.ds(0, LAT)],
                    cnt_out.at[cid, pl.ds(sid * ZR, ZR), :])


def _sc_entry(z_hbm, ids_hbm, part_out, cnt_out,
              b0, b1, b2, b3, trows_v, idx_all, tidx_v, zrow_v, cntv_v,
              acc_s, isem, l0, l1, l2, l3, s0, s1, s2, s3):
    _sc_body(z_hbm, ids_hbm, part_out, cnt_out,
             (b0, b1, b2, b3), trows_v, idx_all, tidx_v, zrow_v, cntv_v,
             acc_s, isem, (l0, l1, l2, l3), (s0, s1, s2, s3))


_sc_segment_sum = functools.partial(
    pl.kernel,
    out_type=(jax.ShapeDtypeStruct((NC, NSEG, LAT), _f32),
              jax.ShapeDtypeStruct((NC, NSEG, LAT), _f32)),
    mesh=plsc.VectorSubcoreMesh(core_axis_name="c", subcore_axis_name="s",
                                num_cores=NC, num_subcores=NS),
    compiler_params=pltpu.CompilerParams(use_tc_tiling_on_sc=False),
    scratch_types=[
        pltpu.VMEM((BLK, ROWW), _f32),
        pltpu.VMEM((BLK, ROWW), _f32),
        pltpu.VMEM((BLK, ROWW), _f32),
        pltpu.VMEM((BLK, ROWW), _f32),
        pltpu.VMEM((TAIL, ROWW), _f32),
        pltpu.VMEM((TRIPS, BLK), jnp.int32),
        pltpu.VMEM((TAIL,), jnp.int32),
        pltpu.VMEM((ZR, ROWW), _f32),
        pltpu.VMEM((ZR, CW), _f32),
        pltpu.VMEM_SHARED((NSEG, ROWW), _f32),
        pltpu.SemaphoreType.DMA,
        pltpu.SemaphoreType.DMA,
        pltpu.SemaphoreType.DMA,
        pltpu.SemaphoreType.DMA,
        pltpu.SemaphoreType.DMA,
        pltpu.SemaphoreType.DMA,
        pltpu.SemaphoreType.DMA,
        pltpu.SemaphoreType.DMA,
        pltpu.SemaphoreType.DMA,
    ],
)(_sc_entry)


def _mlp_body(part_ref, cnt_ref, w1_ref, b1_ref, w2_ref, b2_ref, w3_ref,
              b3_ref, out_ref):
    summed = part_ref[0] + part_ref[1]
    counts = cnt_ref[0] + cnt_ref[1]
    pooled = summed / counts
    h = jnp.dot(pooled, w1_ref[...], preferred_element_type=_f32) + b1_ref[...]
    h = h * (1.0 / (1.0 + jnp.exp(-h)))
    h = jnp.dot(h, w2_ref[...], preferred_element_type=_f32) + b2_ref[...]
    h = h * (1.0 / (1.0 + jnp.exp(-h)))
    out = jnp.sum(h * w3_ref[...], axis=1) + b3_ref[0, 0]
    out_ref[...] = out


_mlp = pl.pallas_call(
    _mlp_body,
    out_shape=jax.ShapeDtypeStruct((NSEG,), _f32),
)


def kernel(z, batch, W1, b1, W2, b2, W3, b3):
    ids = batch.astype(jnp.int32)
    part, cnt = _sc_segment_sum(z, ids)
    return _mlp(part, cnt, W1, b1.reshape(1, LAT), W2, b2.reshape(1, 64),
                W3.reshape(1, 64), b3.reshape(1, 1))


# speedup vs baseline: 6.0746x; 1.1346x over previous
import functools

import jax
import jax.numpy as jnp
from jax import lax
from jax.experimental import pallas as pl
from jax.experimental.pallas import tpu as pltpu
import jax.experimental.pallas.tpu_sc as plsc

N = 100000
NSEG = 1024
LAT = 128
NC, NS = 2, 16
NW = NC * NS
BLK = 128
NFULL = N // BLK
TAIL = N - NFULL * BLK
TRIPS = 25
BASE_Q, BASE_R = divmod(NFULL, NW)
ZR = NSEG // NS
CW = 16
NBUF = 4
LOOKAHEAD = 3

_f32 = jnp.float32


def _sc_body(z_hbm, ids_hbm, part_out, cnt_out,
             bufs, trows_v, idx_all, tidx_v, ones_v, zrow_v, zc_v, cntv_v,
             acc_s, cnt_s,
             isem, lsems, ssems, csem):
    cid = lax.axis_index("c")
    sid = lax.axis_index("s")
    wid = sid * NC + cid
    nb = jnp.where(wid < BASE_R, BASE_Q + 1, BASE_Q)
    base = wid * BASE_Q + jnp.minimum(wid, BASE_R)

    def _fill_rows(i, _):
        zrow_v[i // 8, pl.ds((i % 8) * 16, 16)] = jnp.zeros((16,), _f32)
        return 0
    lax.fori_loop(0, ZR * 8, _fill_rows, 0)

    def _fill_zc(i, _):
        zc_v[i, :] = jnp.zeros((16,), _f32)
        return 0
    lax.fori_loop(0, ZR, _fill_zc, 0)

    def _fill_ones(i, _):
        ones_v[i, :] = jnp.ones((16,), _f32)
        return 0
    lax.fori_loop(0, BLK, _fill_ones, 0)

    base_off = pl.multiple_of(base * BLK, BLK)

    def _load_desc(t, j):
        off = pl.multiple_of(base_off + t * BLK, 8)
        return pltpu.make_async_copy(z_hbm.at[pl.ds(off, BLK), :],
                                     bufs[j], lsems[j])

    def _scat_desc(t, j):
        return pltpu.make_async_copy(bufs[j], acc_s.at[idx_all.at[t]],
                                     ssems[j])

    def _cnt_desc(t):
        return pltpu.make_async_copy(ones_v, cnt_s.at[idx_all.at[t]], csem)

    ids_desc = [
        pltpu.make_async_copy(
            ids_hbm.at[pl.ds(pl.multiple_of(base_off + t * BLK, 8), BLK)],
            idx_all.at[t], isem)
        for t in range(TRIPS)
    ]
    for t in range(TRIPS):
        ids_desc[t].start()
    for t in range(LOOKAHEAD):
        _load_desc(t, t % NBUF).start()

    pltpu.sync_copy(zrow_v, acc_s.at[pl.ds(sid * ZR, ZR), :])
    pltpu.sync_copy(zc_v, cnt_s.at[pl.ds(sid * ZR, ZR), :])
    plsc.subcore_barrier()
    for t in range(TRIPS):
        ids_desc[t].wait()

    NGRP = (TRIPS + NBUF) // NBUF + (1 if (TRIPS + NBUF) % NBUF else 0)

    def _group(g, _):
        t0 = g * NBUF
        for j in range(NBUF):
            t = t0 + j

            @pl.when(jnp.logical_and(t >= 1, t - 1 < nb))
            def _():
                _scat_desc(t - 1, (j - 1) % NBUF).wait()

            u = t + LOOKAHEAD

            @pl.when(u < nb)
            def _():
                _load_desc(u, (j + LOOKAHEAD) % NBUF).start()

            @pl.when(t < nb)
            def _():
                _load_desc(t, j).wait()
                _cnt_desc(t).start(add=True)
                _scat_desc(t, j).start(add=True)

        return 0

    lax.fori_loop(0, NGRP, _group, 0)

    def _cnt_drain(t, _):
        _cnt_desc(0).wait()
        return 0

    lax.fori_loop(0, nb, _cnt_drain, 0)

    @pl.when(wid == NW - 1)
    def _tail():
        pltpu.sync_copy(ids_hbm.at[pl.ds(NFULL * BLK, TAIL)], tidx_v)
        pltpu.sync_copy(z_hbm.at[pl.ds(NFULL * BLK, TAIL), :], trows_v)
        pltpu.sync_copy(trows_v, acc_s.at[tidx_v], add=True)
        pltpu.sync_copy(ones_v.at[pl.ds(0, TAIL), :], cnt_s.at[tidx_v],
                        add=True)

    plsc.subcore_barrier()

    pltpu.sync_copy(acc_s.at[pl.ds(sid * ZR, ZR), :],
                    part_out.at[cid, pl.ds(sid * ZR, ZR), :])
    pltpu.sync_copy(cnt_s.at[pl.ds(sid * ZR, ZR), :], cntv_v)

    def _bcast(r, _):
        row = jnp.full((16,), cntv_v[r, :][0], _f32)
        for k in range(LAT // 16):
            zrow_v[r, pl.ds(k * 16, 16)] = row
        return 0

    lax.fori_loop(0, ZR, _bcast, 0)
    pltpu.sync_copy(zrow_v, cnt_out.at[cid, pl.ds(sid * ZR, ZR), :])


def _sc_entry(z_hbm, ids_hbm, part_out, cnt_out,
              b0, b1, b2, b3, trows_v, idx_all, tidx_v, ones_v, zrow_v,
              zc_v, cntv_v, acc_s, cnt_s, isem, l0, l1, l2, l3, s0, s1,
              s2, s3, csem):
    _sc_body(z_hbm, ids_hbm, part_out, cnt_out,
             (b0, b1, b2, b3), trows_v, idx_all, tidx_v, ones_v, zrow_v,
             zc_v, cntv_v, acc_s, cnt_s,
             isem, (l0, l1, l2, l3), (s0, s1, s2, s3), csem)


_sc_segment_sum = functools.partial(
    pl.kernel,
    out_type=(jax.ShapeDtypeStruct((NC, NSEG, LAT), _f32),
              jax.ShapeDtypeStruct((NC, NSEG, LAT), _f32)),
    mesh=plsc.VectorSubcoreMesh(core_axis_name="c", subcore_axis_name="s",
                                num_cores=NC, num_subcores=NS),
    compiler_params=pltpu.CompilerParams(use_tc_tiling_on_sc=False),
    scratch_types=[
        pltpu.VMEM((BLK, LAT), _f32),
        pltpu.VMEM((BLK, LAT), _f32),
        pltpu.VMEM((BLK, LAT), _f32),
        pltpu.VMEM((BLK, LAT), _f32),
        pltpu.VMEM((TAIL, LAT), _f32),
        pltpu.VMEM((TRIPS, BLK), jnp.int32),
        pltpu.VMEM((TAIL,), jnp.int32),
        pltpu.VMEM((BLK, CW), _f32),
        pltpu.VMEM((ZR, LAT), _f32),
        pltpu.VMEM((ZR, CW), _f32),
        pltpu.VMEM((ZR, CW), _f32),
        pltpu.VMEM_SHARED((NSEG, LAT), _f32),
        pltpu.VMEM_SHARED((NSEG, CW), _f32),
        pltpu.SemaphoreType.DMA,
        pltpu.SemaphoreType.DMA,
        pltpu.SemaphoreType.DMA,
        pltpu.SemaphoreType.DMA,
        pltpu.SemaphoreType.DMA,
        pltpu.SemaphoreType.DMA,
        pltpu.SemaphoreType.DMA,
        pltpu.SemaphoreType.DMA,
        pltpu.SemaphoreType.DMA,
        pltpu.SemaphoreType.DMA,
    ],
)(_sc_entry)


def _mlp_body(part_ref, cnt_ref, w1_ref, b1_ref, w2_ref, b2_ref, w3_ref,
              b3_ref, out_ref):
    summed = part_ref[0] + part_ref[1]
    counts = cnt_ref[0] + cnt_ref[1]
    pooled = summed / counts
    h = jnp.dot(pooled, w1_ref[...], preferred_element_type=_f32) + b1_ref[...]
    h = h * (1.0 / (1.0 + jnp.exp(-h)))
    h = jnp.dot(h, w2_ref[...], preferred_element_type=_f32) + b2_ref[...]
    h = h * (1.0 / (1.0 + jnp.exp(-h)))
    out = jnp.sum(h * w3_ref[...], axis=1) + b3_ref[0, 0]
    out_ref[...] = out


_mlp = pl.pallas_call(
    _mlp_body,
    out_shape=jax.ShapeDtypeStruct((NSEG,), _f32),
)


def kernel(z, batch, W1, b1, W2, b2, W3, b3):
    ids = batch.astype(jnp.int32)
    part, cnt = _sc_segment_sum(z, ids)
    return _mlp(part, cnt, W1, b1.reshape(1, LAT), W2, b2.reshape(1, 64),
                W3.reshape(1, 64), b3.reshape(1, 1))
